# baseline (speedup 1.0000x reference)
import functools

import jax
import jax.numpy as jnp
from jax import lax
from jax.experimental import pallas as pl
from jax.experimental.pallas import tpu as pltpu
from jax.experimental.pallas import tpu_sc as plsc

D = H = W = 128
N = 8192
WIN = 10
NTILE = 32
SLABZ = D // NTILE
SLABW = SLABZ * H * W
NVREG = 7


def _prep_body(cen_ref, quat_ref, sc_ref, den_ref, f_ref, i_ref):
    cz = cen_ref[0:1, :] * (D - 1.0)
    cy = cen_ref[1:2, :] * (H - 1.0)
    cx = cen_ref[2:3, :] * (W - 1.0)

    qw = quat_ref[0:1, :]
    qx = quat_ref[1:2, :]
    qy = quat_ref[2:3, :]
    qz = quat_ref[3:4, :]
    norm = jnp.sqrt(qw * qw + qx * qx + qy * qy + qz * qz) + 1e-08
    w = qw / norm
    x = qx / norm
    y = qy / norm
    z = qz / norm

    r00 = 1 - 2 * (y * y + z * z)
    r01 = 2 * (x * y - z * w)
    r02 = 2 * (x * z + y * w)
    r10 = 2 * (x * y + z * w)
    r11 = 1 - 2 * (x * x + z * z)
    r12 = 2 * (y * z - x * w)
    r20 = 2 * (x * z - y * w)
    r21 = 2 * (y * z + x * w)
    r22 = 1 - 2 * (x * x + y * y)

    s0 = 0.008 + sc_ref[0:1, :] * 0.015
    s1 = 0.008 + sc_ref[1:2, :] * 0.015
    s2 = 0.008 + sc_ref[2:3, :] * 0.015
    i0 = 1.0 / (s0 * s0)
    i1 = 1.0 / (s1 * s1)
    i2 = 1.0 / (s2 * s2)

    sc = -0.5 / 4096.0
    f_ref[0:1, :] = (r00 * r00 * i0 + r01 * r01 * i1 + r02 * r02 * i2) * sc
    f_ref[1:2, :] = (r10 * r10 * i0 + r11 * r11 * i1 + r12 * r12 * i2) * sc
    f_ref[2:3, :] = (r20 * r20 * i0 + r21 * r21 * i1 + r22 * r22 * i2) * sc
    f_ref[3:4, :] = (r00 * r10 * i0 + r01 * r11 * i1 + r02 * r12 * i2) * (2.0 * sc)
    f_ref[4:5, :] = (r00 * r20 * i0 + r01 * r21 * i1 + r02 * r22 * i2) * (2.0 * sc)
    f_ref[5:6, :] = (r10 * r20 * i0 + r11 * r21 * i1 + r12 * r22 * i2) * (2.0 * sc)
    f_ref[6:7, :] = cz
    f_ref[7:8, :] = cy
    f_ref[8:9, :] = cx
    f_ref[9:10, :] = den_ref[0:1, :]

    z0 = jnp.clip(jnp.floor(cz).astype(jnp.int32) - 4, 0, D - WIN)
    y0 = jnp.clip(jnp.floor(cy).astype(jnp.int32) - 4, 0, H - WIN)
    x0 = jnp.clip(jnp.floor(cx).astype(jnp.int32) - 4, 0, W - WIN)
    packed = z0 * (H * W) + y0 * W + x0
    f_ref[10:11, :] = lax.bitcast_convert_type(packed, jnp.float32)
    f_ref[11:12, :] = y0.astype(jnp.float32)
    f_ref[12:13, :] = x0.astype(jnp.float32)
    zero = cz * 0.0
    f_ref[13:14, :] = zero
    f_ref[14:15, :] = zero
    f_ref[15:16, :] = zero
    i_ref[0:1, :] = z0


def _sc_body(params_hbm, z0_hbm, out_hbm, vol_v, zv, wl, pstage, sem0, sem1):
    sems = (sem0, sem1)
    i32 = jnp.int32
    f32 = jnp.float32
    wid = lax.axis_index("s") * 2 + lax.axis_index("c")
    lo = wid * SLABZ

    yoffs, xoffs, idxcs, padms = [], [], [], []
    for v in range(NVREG):
        p = lax.iota(i32, 16) + (16 * v)
        j = p // 10
        l = p % 10
        padm = p < 100
        yoffs.append(j.astype(f32))
        xoffs.append(l.astype(f32))
        idxcs.append(jnp.where(padm, j * W + l, 0))
        padms.append(padm)

    zero16 = jnp.zeros((16,), f32)

    def zbody(i, c):
        vol_v[pl.ds(i * 16, 16)] = zero16
        return c

    lax.fori_loop(0, SLABW // 16, zbody, 0)

    pltpu.sync_copy(z0_hbm, zv)

    def rbody(i, cnt):
        z0v = zv[pl.ds(i * 16, 16)]
        m = (z0v >= lo - (WIN - 1)) & (z0v <= lo + (SLABZ - 1))
        cs = plsc.cumsum(m.astype(i32))
        posv = cs + (cnt - 1)
        plsc.store_scatter(wl, [posv], lax.iota(i32, 16) + i * 16, mask=m)
        return cnt + jnp.max(cs)

    cnt = lax.fori_loop(0, N // 16, rbody, 0)
    plsc.store_scatter(wl, [lax.iota(i32, 16) + cnt], jnp.full((16,), N, i32))
    nch = (cnt + 15) // 16

    def dma(ci, b):
        gidv = wl[pl.ds(ci * 16, 16)]
        return pltpu.make_async_copy(params_hbm.at[gidv], pstage.at[b],
                                     sems[b])

    @pl.when(nch > 0)
    def _():
        dma(0, 0).start()

    def cpair(cp, c):
        for b in range(2):
            ci = cp * 2 + b

            @pl.when(ci < nch)
            def _():
                @pl.when(ci + 1 < nch)
                def _():
                    dma(ci + 1, 1 - b).start()
                dma(ci, b).wait()

                def gbody(g, c2):
                    row = pstage[b, g, :]

                    def sp(k):
                        return row.at[jnp.full((16,), k, i32)].get(
                            mode="promise_in_bounds")

                    caa, cbb, ccc = sp(0), sp(1), sp(2)
                    cab, cac, cbc = sp(3), sp(4), sp(5)
                    czc, cyc, cxc = sp(6), sp(7), sp(8)
                    dens = sp(9)
                    pks = jnp.max(plsc.bitcast(sp(10), i32))
                    z0s = pks >> 14
                    fb = pks & ((1 << 14) - 1)
                    ybase = sp(11) - cyc
                    xbase = sp(12) - cxc
                    zlo = jnp.maximum(z0s, lo)
                    zhi = jnp.minimum(z0s + WIN, lo + SLABZ)

                    dyv, dxv, pre, idxg = [], [], [], []
                    for v in range(NVREG):
                        dy = ybase + yoffs[v]
                        dx = xbase + xoffs[v]
                        dyv.append(dy)
                        dxv.append(dx)
                        pre.append(cbb * dy * dy + ccc * dx * dx
                                   + cbc * dy * dx)
                        idxg.append(idxcs[v] + fb)

                    def pbody(z, c3):
                        dz = jnp.broadcast_to(z, (16,)).astype(f32) - czc
                        zq = caa * dz * dz
                        zy = cab * dz
                        zx = cac * dz
                        pbz = (z - lo) * (H * W)
                        for v in range(NVREG):
                            m = (zq + pre[v]) + zy * dyv[v] + zx * dxv[v]
                            wv = jnp.exp(m) * dens
                            wv = jnp.where(m >= -4.5, wv, 0.0)
                            plsc.addupdate_scatter(
                                vol_v, [idxg[v] + pbz], wv, mask=padms[v])
                        return c3

                    lax.fori_loop(zlo, zhi, pbody, 0)
                    return c2

                lax.fori_loop(0, 16, gbody, 0)
        return c

    lax.fori_loop(0, (nch + 1) // 2, cpair, 0)

    pltpu.sync_copy(vol_v, out_hbm.at[pl.ds(wid * SLABW, SLABW)])


@jax.jit
def kernel(centers, quaternions, scales, density):
    cen_t = centers.T.reshape(3, N)
    quat_t = quaternions.T.reshape(4, N)
    sc_t = scales.T.reshape(3, N)
    den_t = density.reshape(1, N)

    fparams, iparams = pl.pallas_call(
        _prep_body,
        out_shape=[
            jax.ShapeDtypeStruct((16, N), jnp.float32),
            jax.ShapeDtypeStruct((1, N), jnp.int32),
        ],
    )(cen_t, quat_t, sc_t, den_t)

    params_nt = jnp.concatenate(
        [fparams.T, jnp.zeros((16, 16), jnp.float32)], axis=0)
    z0r = iparams.reshape(N)

    mesh = plsc.VectorSubcoreMesh(core_axis_name="c", subcore_axis_name="s")
    volume_flat = pl.kernel(
        _sc_body,
        out_type=jax.ShapeDtypeStruct((D * H * W,), jnp.float32),
        mesh=mesh,
        scratch_types=[
            pltpu.VMEM((SLABW,), jnp.float32),
            pltpu.VMEM((N,), jnp.int32),
            pltpu.VMEM((N + 16,), jnp.int32),
            pltpu.VMEM((2, 16, 16), jnp.float32),
            pltpu.SemaphoreType.DMA,
            pltpu.SemaphoreType.DMA,
        ],
        compiler_params=pltpu.CompilerParams(
            needs_layout_passes=False, use_tc_tiling_on_sc=False),
    )(params_nt, z0r)
    return volume_flat.reshape(D, H, W)


# speedup vs baseline: 1.1080x; 1.1080x over previous
import functools

import jax
import jax.numpy as jnp
from jax import lax
from jax.experimental import pallas as pl
from jax.experimental.pallas import tpu as pltpu
from jax.experimental.pallas import tpu_sc as plsc

D = H = W = 128
N = 8192
WIN = 10
NTILE = 32
SLABZ = D // NTILE
SLABW = SLABZ * H * W
NVREG = 7


def _prep_body(cen_ref, quat_ref, sc_ref, den_ref, f_ref, i_ref):
    cz = cen_ref[0:1, :] * (D - 1.0)
    cy = cen_ref[1:2, :] * (H - 1.0)
    cx = cen_ref[2:3, :] * (W - 1.0)

    qw = quat_ref[0:1, :]
    qx = quat_ref[1:2, :]
    qy = quat_ref[2:3, :]
    qz = quat_ref[3:4, :]
    norm = jnp.sqrt(qw * qw + qx * qx + qy * qy + qz * qz) + 1e-08
    w = qw / norm
    x = qx / norm
    y = qy / norm
    z = qz / norm

    r00 = 1 - 2 * (y * y + z * z)
    r01 = 2 * (x * y - z * w)
    r02 = 2 * (x * z + y * w)
    r10 = 2 * (x * y + z * w)
    r11 = 1 - 2 * (x * x + z * z)
    r12 = 2 * (y * z - x * w)
    r20 = 2 * (x * z - y * w)
    r21 = 2 * (y * z + x * w)
    r22 = 1 - 2 * (x * x + y * y)

    s0 = 0.008 + sc_ref[0:1, :] * 0.015
    s1 = 0.008 + sc_ref[1:2, :] * 0.015
    s2 = 0.008 + sc_ref[2:3, :] * 0.015
    i0 = 1.0 / (s0 * s0)
    i1 = 1.0 / (s1 * s1)
    i2 = 1.0 / (s2 * s2)

    sc = -0.5 / 4096.0
    f_ref[0:1, :] = (r00 * r00 * i0 + r01 * r01 * i1 + r02 * r02 * i2) * sc
    f_ref[1:2, :] = (r10 * r10 * i0 + r11 * r11 * i1 + r12 * r12 * i2) * sc
    f_ref[2:3, :] = (r20 * r20 * i0 + r21 * r21 * i1 + r22 * r22 * i2) * sc
    f_ref[3:4, :] = (r00 * r10 * i0 + r01 * r11 * i1 + r02 * r12 * i2) * (2.0 * sc)
    f_ref[4:5, :] = (r00 * r20 * i0 + r01 * r21 * i1 + r02 * r22 * i2) * (2.0 * sc)
    f_ref[5:6, :] = (r10 * r20 * i0 + r11 * r21 * i1 + r12 * r22 * i2) * (2.0 * sc)
    f_ref[6:7, :] = cz
    f_ref[7:8, :] = cy
    f_ref[8:9, :] = cx
    f_ref[9:10, :] = den_ref[0:1, :]

    z0 = jnp.clip(jnp.floor(cz).astype(jnp.int32) - 4, 0, D - WIN)
    y0 = jnp.clip(jnp.floor(cy).astype(jnp.int32) - 4, 0, H - WIN)
    x0 = jnp.clip(jnp.floor(cx).astype(jnp.int32) - 4, 0, W - WIN)
    packed = z0 * (H * W) + y0 * W + x0
    f_ref[10:11, :] = lax.bitcast_convert_type(packed, jnp.float32)
    f_ref[11:12, :] = y0.astype(jnp.float32)
    f_ref[12:13, :] = x0.astype(jnp.float32)
    zero = cz * 0.0
    f_ref[13:14, :] = zero
    f_ref[14:15, :] = zero
    f_ref[15:16, :] = zero
    i_ref[0:1, :] = z0


def _sc_body(params_hbm, z0_hbm, out_hbm, vol_v, zv, wl, pstage, sem0, sem1):
    sems = (sem0, sem1)
    i32 = jnp.int32
    f32 = jnp.float32
    wid = lax.axis_index("s") * 2 + lax.axis_index("c")
    lo = wid * SLABZ

    yoffs, xoffs, idxcs, padms = [], [], [], []
    for v in range(NVREG):
        p = lax.iota(i32, 16) + (16 * v)
        j = p // 10
        l = p % 10
        padm = p < 100
        yoffs.append(j.astype(f32))
        xoffs.append(l.astype(f32))
        idxcs.append(jnp.where(padm, j * W + l, 0))
        padms.append(padm)

    zdma = pltpu.make_async_copy(z0_hbm, zv, sem1)
    zdma.start()

    zero16 = jnp.zeros((16,), f32)

    def zbody(i, c):
        for u in range(4):
            vol_v[pl.ds(i * 64 + u * 16, 16)] = zero16
        return c

    lax.fori_loop(0, SLABW // 64, zbody, 0)
    zdma.wait()

    def rbody(i, cntv):
        z0v = zv[pl.ds(i * 16, 16)]
        m = (z0v >= lo - (WIN - 1)) & (z0v <= lo + (SLABZ - 1))
        cs = plsc.cumsum(jnp.where(m, 1, 0))
        posv = cs - 1 + cntv
        plsc.store_scatter(wl, [posv], lax.iota(i32, 16) + i * 16, mask=m)
        return cntv + plsc.all_reduce_population_count(m)

    cntv = lax.fori_loop(0, N // 16, rbody, jnp.zeros((16,), i32))
    cnt = jnp.max(cntv)
    plsc.store_scatter(wl, [lax.iota(i32, 16) + cnt], jnp.full((16,), N, i32))
    nch = (cnt + 15) // 16

    def dma(ci, b):
        gidv = wl[pl.ds(ci * 16, 16)]
        return pltpu.make_async_copy(params_hbm.at[gidv], pstage.at[b],
                                     sems[b])

    @pl.when(nch > 0)
    def _():
        dma(0, 0).start()

    def cpair(cp, c):
        for b in range(2):
            ci = cp * 2 + b

            @pl.when(ci < nch)
            def _():
                @pl.when(ci + 1 < nch)
                def _():
                    dma(ci + 1, 1 - b).start()
                dma(ci, b).wait()

                def gbody(g, c2):
                    row = pstage[b, g, :]

                    def sp(k):
                        return row.at[jnp.full((16,), k, i32)].get(
                            mode="promise_in_bounds")

                    caa, cbb, ccc = sp(0), sp(1), sp(2)
                    cab, cac, cbc = sp(3), sp(4), sp(5)
                    czc, cyc, cxc = sp(6), sp(7), sp(8)
                    dens = sp(9)
                    pks = jnp.max(plsc.bitcast(sp(10), i32))
                    z0s = pks >> 14
                    fb = pks & ((1 << 14) - 1)
                    ybase = sp(11) - cyc
                    xbase = sp(12) - cxc
                    zlo = jnp.maximum(z0s, lo)
                    zhi = jnp.minimum(z0s + WIN, lo + SLABZ)

                    dyv, dxv, pre, idxg = [], [], [], []
                    for v in range(NVREG):
                        dy = ybase + yoffs[v]
                        dx = xbase + xoffs[v]
                        dyv.append(dy)
                        dxv.append(dx)
                        pre.append(cbb * dy * dy + ccc * dx * dx
                                   + cbc * dy * dx)
                        idxg.append(idxcs[v] + fb)

                    def pbody(z, c3):
                        dz = jnp.broadcast_to(z, (16,)).astype(f32) - czc
                        zq = caa * dz * dz
                        zy = cab * dz
                        zx = cac * dz
                        pbz = (z - lo) * (H * W)
                        for v in range(NVREG):
                            m = (zq + pre[v]) + zy * dyv[v] + zx * dxv[v]
                            wv = jnp.exp(m) * dens
                            wv = jnp.where(m >= -4.5, wv, 0.0)
                            plsc.addupdate_scatter(
                                vol_v, [idxg[v] + pbz], wv, mask=padms[v])
                        return c3

                    lax.fori_loop(zlo, zhi, pbody, 0)
                    return c2

                lax.fori_loop(0, 16, gbody, 0)
        return c

    lax.fori_loop(0, (nch + 1) // 2, cpair, 0)

    pltpu.sync_copy(vol_v, out_hbm.at[pl.ds(wid * SLABW, SLABW)])


@jax.jit
def kernel(centers, quaternions, scales, density):
    cen_t = centers.T.reshape(3, N)
    quat_t = quaternions.T.reshape(4, N)
    sc_t = scales.T.reshape(3, N)
    den_t = density.reshape(1, N)

    fparams, iparams = pl.pallas_call(
        _prep_body,
        out_shape=[
            jax.ShapeDtypeStruct((16, N), jnp.float32),
            jax.ShapeDtypeStruct((1, N), jnp.int32),
        ],
    )(cen_t, quat_t, sc_t, den_t)

    params_nt = jnp.concatenate(
        [fparams.T, jnp.zeros((16, 16), jnp.float32)], axis=0)
    z0r = iparams.reshape(N)

    mesh = plsc.VectorSubcoreMesh(core_axis_name="c", subcore_axis_name="s")
    volume_flat = pl.kernel(
        _sc_body,
        out_type=jax.ShapeDtypeStruct((D * H * W,), jnp.float32),
        mesh=mesh,
        scratch_types=[
            pltpu.VMEM((SLABW,), jnp.float32),
            pltpu.VMEM((N,), jnp.int32),
            pltpu.VMEM((N + 16,), jnp.int32),
            pltpu.VMEM((2, 16, 16), jnp.float32),
            pltpu.SemaphoreType.DMA,
            pltpu.SemaphoreType.DMA,
        ],
        compiler_params=pltpu.CompilerParams(
            needs_layout_passes=False, use_tc_tiling_on_sc=False),
    )(params_nt, z0r)
    return volume_flat.reshape(D, H, W)


# speedup vs baseline: 1.1390x; 1.0280x over previous
import functools

import jax
import jax.numpy as jnp
from jax import lax
from jax.experimental import pallas as pl
from jax.experimental.pallas import tpu as pltpu
from jax.experimental.pallas import tpu_sc as plsc

D = H = W = 128
N = 8192
WIN = 10
NTILE = 32
SLABZ = D // NTILE
SLABW = SLABZ * H * W
NVREG = 7


def _prep_body(cen_ref, quat_ref, sc_ref, den_ref, f_ref, i_ref):
    cz = cen_ref[0:1, :] * (D - 1.0)
    cy = cen_ref[1:2, :] * (H - 1.0)
    cx = cen_ref[2:3, :] * (W - 1.0)

    qw = quat_ref[0:1, :]
    qx = quat_ref[1:2, :]
    qy = quat_ref[2:3, :]
    qz = quat_ref[3:4, :]
    norm = jnp.sqrt(qw * qw + qx * qx + qy * qy + qz * qz) + 1e-08
    w = qw / norm
    x = qx / norm
    y = qy / norm
    z = qz / norm

    r00 = 1 - 2 * (y * y + z * z)
    r01 = 2 * (x * y - z * w)
    r02 = 2 * (x * z + y * w)
    r10 = 2 * (x * y + z * w)
    r11 = 1 - 2 * (x * x + z * z)
    r12 = 2 * (y * z - x * w)
    r20 = 2 * (x * z - y * w)
    r21 = 2 * (y * z + x * w)
    r22 = 1 - 2 * (x * x + y * y)

    s0 = 0.008 + sc_ref[0:1, :] * 0.015
    s1 = 0.008 + sc_ref[1:2, :] * 0.015
    s2 = 0.008 + sc_ref[2:3, :] * 0.015
    i0 = 1.0 / (s0 * s0)
    i1 = 1.0 / (s1 * s1)
    i2 = 1.0 / (s2 * s2)

    sc = -0.5 / 4096.0
    f_ref[0:1, :] = (r00 * r00 * i0 + r01 * r01 * i1 + r02 * r02 * i2) * sc
    f_ref[1:2, :] = (r10 * r10 * i0 + r11 * r11 * i1 + r12 * r12 * i2) * sc
    f_ref[2:3, :] = (r20 * r20 * i0 + r21 * r21 * i1 + r22 * r22 * i2) * sc
    f_ref[3:4, :] = (r00 * r10 * i0 + r01 * r11 * i1 + r02 * r12 * i2) * (2.0 * sc)
    f_ref[4:5, :] = (r00 * r20 * i0 + r01 * r21 * i1 + r02 * r22 * i2) * (2.0 * sc)
    f_ref[5:6, :] = (r10 * r20 * i0 + r11 * r21 * i1 + r12 * r22 * i2) * (2.0 * sc)
    f_ref[6:7, :] = cz
    f_ref[7:8, :] = cy
    f_ref[8:9, :] = cx
    f_ref[9:10, :] = den_ref[0:1, :]

    z0 = jnp.clip(jnp.floor(cz).astype(jnp.int32) - 4, 0, D - WIN)
    y0 = jnp.clip(jnp.floor(cy).astype(jnp.int32) - 4, 0, H - WIN)
    x0 = jnp.clip(jnp.floor(cx).astype(jnp.int32) - 4, 0, W - WIN)
    packed = z0 * (H * W) + y0 * W + x0
    f_ref[10:11, :] = lax.bitcast_convert_type(packed, jnp.float32)
    f_ref[11:12, :] = y0.astype(jnp.float32)
    f_ref[12:13, :] = x0.astype(jnp.float32)
    zero = cz * 0.0
    f_ref[13:14, :] = zero
    f_ref[14:15, :] = zero
    f_ref[15:16, :] = zero
    i_ref[0:1, :] = z0


def _sc_body(params_hbm, z0_hbm, out_hbm, vol_v, zv, wl, pstage, sem0, sem1):
    sems = (sem0, sem1)
    i32 = jnp.int32
    f32 = jnp.float32
    wid = lax.axis_index("s") * 2 + lax.axis_index("c")
    lo = wid * SLABZ

    yoffs, xoffs, idxcs, padms = [], [], [], []
    for v in range(NVREG):
        p = lax.iota(i32, 16) + (16 * v)
        j = p // 10
        l = p % 10
        padm = p < 100
        yoffs.append(j.astype(f32))
        xoffs.append(l.astype(f32))
        idxcs.append(jnp.where(padm, j * W + l, 0))
        padms.append(padm)

    zdma = pltpu.make_async_copy(z0_hbm, zv, sem1)
    zdma.start()

    zero16 = jnp.zeros((16,), f32)

    def zbody(i, c):
        for u in range(4):
            vol_v[pl.ds(i * 64 + u * 16, 16)] = zero16
        return c

    lax.fori_loop(0, SLABW // 64, zbody, 0)
    zdma.wait()

    def rbody(i, cntv):
        z0v = zv[pl.ds(i * 16, 16)]
        m = (z0v >= lo - (WIN - 1)) & (z0v <= lo + (SLABZ - 1))
        cs = plsc.cumsum(jnp.where(m, 1, 0))
        posv = cs - 1 + cntv
        plsc.store_scatter(wl, [posv], lax.iota(i32, 16) + i * 16, mask=m)
        return cntv + plsc.all_reduce_population_count(m)

    cntv = lax.fori_loop(0, N // 16, rbody, jnp.zeros((16,), i32))
    cnt = jnp.max(cntv)
    plsc.store_scatter(wl, [lax.iota(i32, 16) + cnt], jnp.full((16,), N, i32))
    nch = (cnt + 15) // 16

    def dma(ci, b):
        gidv = wl[pl.ds(ci * 16, 16)]
        return pltpu.make_async_copy(params_hbm.at[gidv], pstage.at[b],
                                     sems[b])

    @pl.when(nch > 0)
    def _():
        dma(0, 0).start()

    def cpair(cp, c):
        for b in range(2):
            ci = cp * 2 + b

            @pl.when(ci < nch)
            def _():
                @pl.when(ci + 1 < nch)
                def _():
                    dma(ci + 1, 1 - b).start()
                dma(ci, b).wait()

                def gbody(g, c2):
                    row = pstage[b, g, :]

                    def sp(k):
                        return row.at[jnp.full((16,), k, i32)].get(
                            mode="promise_in_bounds")

                    caa, cbb, ccc = sp(0), sp(1), sp(2)
                    cab, cac, cbc = sp(3), sp(4), sp(5)
                    czc, cyc, cxc = sp(6), sp(7), sp(8)
                    dens = sp(9)
                    pks = plsc.bitcast(row, i32)[10]
                    z0s = pks >> 14
                    fb = pks & ((1 << 14) - 1)
                    ybase = sp(11) - cyc
                    xbase = sp(12) - cxc
                    zlo = jnp.maximum(z0s, lo)
                    zhi = jnp.minimum(z0s + WIN, lo + SLABZ)

                    dyv, dxv, pre, idxg = [], [], [], []
                    for v in range(NVREG):
                        dy = ybase + yoffs[v]
                        dx = xbase + xoffs[v]
                        dyv.append(dy)
                        dxv.append(dx)
                        pre.append(cbb * dy * dy + ccc * dx * dx
                                   + cbc * dy * dx)
                        idxg.append(idxcs[v] + fb)

                    def pbody(z, c3):
                        dz = jnp.broadcast_to(z, (16,)).astype(f32) - czc
                        zq = caa * dz * dz
                        zy = cab * dz
                        zx = cac * dz
                        pbz = (z - lo) * (H * W)
                        for v in range(NVREG):
                            m = (zq + pre[v]) + zy * dyv[v] + zx * dxv[v]
                            wv = jnp.exp(m) * dens
                            wv = jnp.where(m >= -4.5, wv, 0.0)
                            plsc.addupdate_scatter(
                                vol_v, [idxg[v] + pbz], wv, mask=padms[v])
                        return c3

                    lax.fori_loop(zlo, zhi, pbody, 0)
                    return c2

                lax.fori_loop(0, 16, gbody, 0)
        return c

    lax.fori_loop(0, (nch + 1) // 2, cpair, 0)

    pltpu.sync_copy(vol_v, out_hbm.at[pl.ds(wid * SLABW, SLABW)])


@jax.jit
def kernel(centers, quaternions, scales, density):
    cen_t = centers.T.reshape(3, N)
    quat_t = quaternions.T.reshape(4, N)
    sc_t = scales.T.reshape(3, N)
    den_t = density.reshape(1, N)

    fparams, iparams = pl.pallas_call(
        _prep_body,
        out_shape=[
            jax.ShapeDtypeStruct((16, N), jnp.float32),
            jax.ShapeDtypeStruct((1, N), jnp.int32),
        ],
    )(cen_t, quat_t, sc_t, den_t)

    params_nt = jnp.concatenate(
        [fparams.T, jnp.zeros((16, 16), jnp.float32)], axis=0)
    z0r = iparams.reshape(N)

    mesh = plsc.VectorSubcoreMesh(core_axis_name="c", subcore_axis_name="s")
    volume_flat = pl.kernel(
        _sc_body,
        out_type=jax.ShapeDtypeStruct((D * H * W,), jnp.float32),
        mesh=mesh,
        scratch_types=[
            pltpu.VMEM((SLABW,), jnp.float32),
            pltpu.VMEM((N,), jnp.int32),
            pltpu.VMEM((N + 16,), jnp.int32),
            pltpu.VMEM((2, 16, 16), jnp.float32),
            pltpu.SemaphoreType.DMA,
            pltpu.SemaphoreType.DMA,
        ],
        compiler_params=pltpu.CompilerParams(
            needs_layout_passes=False, use_tc_tiling_on_sc=False),
    )(params_nt, z0r)
    return volume_flat.reshape(D, H, W)


# speedup vs baseline: 1.1751x; 1.0317x over previous
import functools

import jax
import jax.numpy as jnp
from jax import lax
from jax.experimental import pallas as pl
from jax.experimental.pallas import tpu as pltpu
from jax.experimental.pallas import tpu_sc as plsc

D = H = W = 128
N = 8192
WIN = 10
NTILE = 32
SLABZ = D // NTILE
SLABW = SLABZ * H * W
NVREG = 7
NPOS = NVREG * 16
PROW = 256


def _prep_body(cen_ref, quat_ref, sc_ref, den_ref, f_ref, i_ref):
    cz = cen_ref[0:1, :] * (D - 1.0)
    cy = cen_ref[1:2, :] * (H - 1.0)
    cx = cen_ref[2:3, :] * (W - 1.0)

    qw = quat_ref[0:1, :]
    qx = quat_ref[1:2, :]
    qy = quat_ref[2:3, :]
    qz = quat_ref[3:4, :]
    norm = jnp.sqrt(qw * qw + qx * qx + qy * qy + qz * qz) + 1e-08
    w = qw / norm
    x = qx / norm
    y = qy / norm
    z = qz / norm

    r00 = 1 - 2 * (y * y + z * z)
    r01 = 2 * (x * y - z * w)
    r02 = 2 * (x * z + y * w)
    r10 = 2 * (x * y + z * w)
    r11 = 1 - 2 * (x * x + z * z)
    r12 = 2 * (y * z - x * w)
    r20 = 2 * (x * z - y * w)
    r21 = 2 * (y * z + x * w)
    r22 = 1 - 2 * (x * x + y * y)

    s0 = 0.008 + sc_ref[0:1, :] * 0.015
    s1 = 0.008 + sc_ref[1:2, :] * 0.015
    s2 = 0.008 + sc_ref[2:3, :] * 0.015
    i0 = 1.0 / (s0 * s0)
    i1 = 1.0 / (s1 * s1)
    i2 = 1.0 / (s2 * s2)

    sc = -0.5 / 4096.0
    f_ref[0:1, :] = (r00 * r00 * i0 + r01 * r01 * i1 + r02 * r02 * i2) * sc
    f_ref[1:2, :] = (r10 * r10 * i0 + r11 * r11 * i1 + r12 * r12 * i2) * sc
    f_ref[2:3, :] = (r20 * r20 * i0 + r21 * r21 * i1 + r22 * r22 * i2) * sc
    f_ref[3:4, :] = (r00 * r10 * i0 + r01 * r11 * i1 + r02 * r12 * i2) * (2.0 * sc)
    f_ref[4:5, :] = (r00 * r20 * i0 + r01 * r21 * i1 + r02 * r22 * i2) * (2.0 * sc)
    f_ref[5:6, :] = (r10 * r20 * i0 + r11 * r21 * i1 + r12 * r22 * i2) * (2.0 * sc)
    f_ref[6:7, :] = cz
    f_ref[7:8, :] = cy
    f_ref[8:9, :] = cx
    f_ref[9:10, :] = den_ref[0:1, :]

    z0 = jnp.clip(jnp.floor(cz).astype(jnp.int32) - 4, 0, D - WIN)
    y0 = jnp.clip(jnp.floor(cy).astype(jnp.int32) - 4, 0, H - WIN)
    x0 = jnp.clip(jnp.floor(cx).astype(jnp.int32) - 4, 0, W - WIN)
    packed = z0 * (H * W) + y0 * W + x0
    f_ref[10:11, :] = lax.bitcast_convert_type(packed, jnp.float32)
    f_ref[11:12, :] = y0.astype(jnp.float32)
    f_ref[12:13, :] = x0.astype(jnp.float32)
    zero = cz * 0.0
    f_ref[13:14, :] = zero
    f_ref[14:15, :] = zero
    f_ref[15:16, :] = zero
    i_ref[0:1, :] = z0

    cbb = f_ref[1:2, :]
    ccc = f_ref[2:3, :]
    cab = f_ref[3:4, :]
    cac = f_ref[4:5, :]
    cbc = f_ref[5:6, :]
    pcol = lax.broadcasted_iota(jnp.int32, (NPOS, 1), 0)
    jcol = (pcol // WIN).astype(jnp.float32)
    lcol = (pcol % WIN).astype(jnp.float32)
    ybase = f_ref[11:12, :] - cy
    xbase = f_ref[12:13, :] - cx
    dyall = ybase + jcol
    dxall = xbase + lcol
    f_ref[16:16 + NPOS, :] = (cbb * dyall * dyall + ccc * dxall * dxall
                              + cbc * dyall * dxall)
    f_ref[16 + NPOS:16 + 2 * NPOS, :] = cab * dyall + cac * dxall
    f_ref[16 + 2 * NPOS:PROW, :] = jnp.zeros((PROW - 16 - 2 * NPOS, 1),
                                             jnp.float32) + zero


def _sc_body(params_hbm, z0_hbm, out_hbm, vol_v, zv, wl, pstage, sem0, sem1):
    sems = (sem0, sem1)
    i32 = jnp.int32
    f32 = jnp.float32
    wid = lax.axis_index("s") * 2 + lax.axis_index("c")
    lo = wid * SLABZ

    idxcs, padms = [], []
    for v in range(NVREG):
        p = lax.iota(i32, 16) + (16 * v)
        j = p // 10
        l = p % 10
        padm = p < 100
        idxcs.append(jnp.where(padm, j * W + l, 0))
        padms.append(padm)

    zdma = pltpu.make_async_copy(z0_hbm, zv, sem1)
    zdma.start()

    zero16 = jnp.zeros((16,), f32)

    def zbody(i, c):
        for u in range(4):
            vol_v[pl.ds(i * 64 + u * 16, 16)] = zero16
        return c

    lax.fori_loop(0, SLABW // 64, zbody, 0)
    zdma.wait()

    def rbody(i, cntv):
        z0v = zv[pl.ds(i * 16, 16)]
        m = (z0v >= lo - (WIN - 1)) & (z0v <= lo + (SLABZ - 1))
        cs = plsc.cumsum(jnp.where(m, 1, 0))
        posv = cs - 1 + cntv
        plsc.store_scatter(wl, [posv], lax.iota(i32, 16) + i * 16, mask=m)
        return cntv + plsc.all_reduce_population_count(m)

    cntv = lax.fori_loop(0, N // 16, rbody, jnp.zeros((16,), i32))
    cnt = jnp.max(cntv)
    plsc.store_scatter(wl, [lax.iota(i32, 16) + cnt], jnp.full((16,), N, i32))
    nch = (cnt + 15) // 16

    def dma(ci, b):
        gidv = wl[pl.ds(ci * 16, 16)]
        return pltpu.make_async_copy(params_hbm.at[gidv], pstage.at[b],
                                     sems[b])

    @pl.when(nch > 0)
    def _():
        dma(0, 0).start()

    def cpair(cp, c):
        for b in range(2):
            ci = cp * 2 + b

            @pl.when(ci < nch)
            def _():
                @pl.when(ci + 1 < nch)
                def _():
                    dma(ci + 1, 1 - b).start()
                dma(ci, b).wait()

                def gbody(g, c2):
                    row = pstage[b, g, pl.ds(0, 16)]

                    def sp(k):
                        return row.at[jnp.full((16,), k, i32)].get(
                            mode="promise_in_bounds")

                    caa = sp(0)
                    czc = sp(6)
                    dens = sp(9)
                    pks = plsc.bitcast(row, i32)[10]
                    z0s = pks >> 14
                    fb = pks & ((1 << 14) - 1)
                    zlo = jnp.maximum(z0s, lo)
                    zhi = jnp.minimum(z0s + WIN, lo + SLABZ)

                    pre, cross, idxg = [], [], []
                    for v in range(NVREG):
                        pre.append(pstage[b, g, pl.ds(16 + 16 * v, 16)])
                        cross.append(
                            pstage[b, g, pl.ds(16 + NPOS + 16 * v, 16)])
                        idxg.append(idxcs[v] + fb)

                    def pbody(z, c3):
                        dz = jnp.broadcast_to(z, (16,)).astype(f32) - czc
                        zq = caa * dz * dz
                        pbz = (z - lo) * (H * W)
                        for v in range(NVREG):
                            m = (zq + pre[v]) + dz * cross[v]
                            wv = jnp.exp(m) * dens
                            wv = jnp.where(m >= -4.5, wv, 0.0)
                            plsc.addupdate_scatter(
                                vol_v, [idxg[v] + pbz], wv, mask=padms[v])
                        return c3

                    lax.fori_loop(zlo, zhi, pbody, 0)
                    return c2

                lax.fori_loop(0, 16, gbody, 0)
        return c

    lax.fori_loop(0, (nch + 1) // 2, cpair, 0)

    pltpu.sync_copy(vol_v, out_hbm.at[pl.ds(wid * SLABW, SLABW)])


@jax.jit
def kernel(centers, quaternions, scales, density):
    cen_t = centers.T.reshape(3, N)
    quat_t = quaternions.T.reshape(4, N)
    sc_t = scales.T.reshape(3, N)
    den_t = density.reshape(1, N)

    fparams, iparams = pl.pallas_call(
        _prep_body,
        out_shape=[
            jax.ShapeDtypeStruct((PROW, N), jnp.float32),
            jax.ShapeDtypeStruct((1, N), jnp.int32),
        ],
    )(cen_t, quat_t, sc_t, den_t)

    params_nt = jnp.concatenate(
        [fparams.T, jnp.zeros((16, PROW), jnp.float32)], axis=0)
    z0r = iparams.reshape(N)

    mesh = plsc.VectorSubcoreMesh(core_axis_name="c", subcore_axis_name="s")
    volume_flat = pl.kernel(
        _sc_body,
        out_type=jax.ShapeDtypeStruct((D * H * W,), jnp.float32),
        mesh=mesh,
        scratch_types=[
            pltpu.VMEM((SLABW,), jnp.float32),
            pltpu.VMEM((N,), jnp.int32),
            pltpu.VMEM((N + 16,), jnp.int32),
            pltpu.VMEM((2, 16, PROW), jnp.float32),
            pltpu.SemaphoreType.DMA,
            pltpu.SemaphoreType.DMA,
        ],
        compiler_params=pltpu.CompilerParams(
            needs_layout_passes=False, use_tc_tiling_on_sc=False),
    )(params_nt, z0r)
    return volume_flat.reshape(D, H, W)
